# Initial kernel scaffold; baseline (speedup 1.0000x reference)
import functools

import jax
import jax.numpy as jnp
from jax import lax
from jax.experimental import pallas as pl
from jax.experimental.pallas import tpu as pltpu
from jax.experimental.pallas import tpu_sc as plsc

N = 10000
E = 320000
D = 128

NC = 2
NS = 16
NW = NC * NS
EPT = E // NW
BLK = 80
NBLK = EPT // BLK
RPT = N // NS
ZCH = 125

_MESH = plsc.VectorSubcoreMesh(core_axis_name="c", subcore_axis_name="s")


@functools.partial(
    pl.kernel,
    out_type=jax.ShapeDtypeStruct((NC, N, 16), jnp.float32),
    mesh=_MESH,
    scratch_types=[
        pltpu.VMEM((NBLK, BLK), jnp.int32),
        pltpu.VMEM((BLK, 16), jnp.float32),
        pltpu.VMEM((ZCH, 16), jnp.float32),
        pltpu.VMEM_SHARED((N, 16), jnp.float32),
    ],
)
def _sc_degree(dst_hbm, ones_hbm, zer_hbm, out_hbm, dst_v, ones_v, zer_v, acc):
    cid = lax.axis_index("c")
    sid = lax.axis_index("s")
    wid = cid * NS + sid
    pltpu.sync_copy(dst_hbm.at[wid], dst_v)
    pltpu.sync_copy(ones_hbm, ones_v)
    pltpu.sync_copy(zer_hbm, zer_v)
    row0 = sid * RPT
    for k in range(RPT // ZCH):
        pltpu.sync_copy(zer_v, acc.at[pl.ds(row0 + k * ZCH, ZCH)])
    plsc.subcore_barrier()

    def body(j, carry):
        pltpu.sync_copy(ones_v, acc.at[dst_v.at[j]], add=True)
        return carry

    lax.fori_loop(0, NBLK, body, 0)
    plsc.subcore_barrier()
    pltpu.sync_copy(acc.at[pl.ds(row0, RPT)], out_hbm.at[cid, pl.ds(row0, RPT)])


@functools.partial(
    pl.kernel,
    out_type=jax.ShapeDtypeStruct((NC, N, D), jnp.float32),
    mesh=_MESH,
    scratch_types=[
        pltpu.VMEM((NBLK, BLK), jnp.int32),
        pltpu.VMEM((NBLK, BLK), jnp.int32),
        pltpu.VMEM((BLK, D), jnp.float32),
        pltpu.VMEM((ZCH, D), jnp.float32),
        pltpu.VMEM_SHARED((N, D), jnp.float32),
        pltpu.SemaphoreType.DMA,
    ],
)
def _sc_scatter(h_hbm, src_hbm, dst_hbm, zer_hbm, out_hbm,
                src_v, dst_v, buf, zer_v, acc, sem):
    cid = lax.axis_index("c")
    sid = lax.axis_index("s")
    wid = cid * NS + sid
    pltpu.sync_copy(src_hbm.at[wid], src_v)
    pltpu.sync_copy(dst_hbm.at[wid], dst_v)
    pltpu.sync_copy(zer_hbm, zer_v)
    row0 = sid * RPT
    for k in range(RPT // ZCH):
        pltpu.sync_copy(zer_v, acc.at[pl.ds(row0 + k * ZCH, ZCH)])
    plsc.subcore_barrier()

    def body(j, carry):
        pltpu.async_copy(h_hbm.at[src_v.at[j]], buf, sem).wait()
        pltpu.sync_copy(buf, acc.at[dst_v.at[j]], add=True)
        return carry

    lax.fori_loop(0, NBLK, body, 0)
    plsc.subcore_barrier()
    pltpu.sync_copy(acc.at[pl.ds(row0, RPT)], out_hbm.at[cid, pl.ds(row0, RPT)])


_RB = 1000
_GRID = N // _RB


def _tc_prescale_body(degp_ref, x_ref, w_ref, hp_ref, dinv_ref):
    d = degp_ref[...]
    deg = d[0, :, 0] + d[1, :, 0] + 1.0
    dinv = lax.rsqrt(deg)
    h = jnp.dot(x_ref[...], w_ref[...], preferred_element_type=jnp.float32)
    hp_ref[...] = h * dinv[:, None]
    dinv_ref[...] = jnp.broadcast_to(dinv[:, None], (_RB, D))


def _tc_prescale(degp, x, w1):
    return pl.pallas_call(
        _tc_prescale_body,
        grid=(_GRID,),
        in_specs=[
            pl.BlockSpec((NC, _RB, 16), lambda i: (0, i, 0)),
            pl.BlockSpec((_RB, D), lambda i: (i, 0)),
            pl.BlockSpec((D, D), lambda i: (0, 0)),
        ],
        out_specs=[
            pl.BlockSpec((_RB, D), lambda i: (i, 0)),
            pl.BlockSpec((_RB, D), lambda i: (i, 0)),
        ],
        out_shape=[
            jax.ShapeDtypeStruct((N, D), jnp.float32),
            jax.ShapeDtypeStruct((N, D), jnp.float32),
        ],
    )(degp, x, w1)


def _tc_mid_body(agg_ref, hp_ref, dinv_ref, b_ref, w_ref, hp2_ref):
    a = agg_ref[...]
    dinv = dinv_ref[...]
    pre = dinv * (a[0] + a[1] + hp_ref[...]) + b_ref[...]
    y = jnp.where(pre > 0, pre, jnp.expm1(pre))
    h2 = jnp.dot(y, w_ref[...], preferred_element_type=jnp.float32)
    hp2_ref[...] = h2 * dinv


def _tc_mid(agg, hp, dinv, b1, w2):
    return pl.pallas_call(
        _tc_mid_body,
        grid=(_GRID,),
        in_specs=[
            pl.BlockSpec((NC, _RB, D), lambda i: (0, i, 0)),
            pl.BlockSpec((_RB, D), lambda i: (i, 0)),
            pl.BlockSpec((_RB, D), lambda i: (i, 0)),
            pl.BlockSpec((1, D), lambda i: (0, 0)),
            pl.BlockSpec((D, D), lambda i: (0, 0)),
        ],
        out_specs=pl.BlockSpec((_RB, D), lambda i: (i, 0)),
        out_shape=jax.ShapeDtypeStruct((N, D), jnp.float32),
    )(agg, hp, dinv, b1, w2)


def _tc_final_body(agg_ref, hp_ref, dinv_ref, b_ref, y_ref):
    a = agg_ref[...]
    pre = dinv_ref[...] * (a[0] + a[1] + hp_ref[...]) + b_ref[...]
    y_ref[...] = jnp.where(pre > 0, pre, jnp.expm1(pre))


def _tc_final(agg, hp, dinv, b2):
    return pl.pallas_call(
        _tc_final_body,
        grid=(_GRID,),
        in_specs=[
            pl.BlockSpec((NC, _RB, D), lambda i: (0, i, 0)),
            pl.BlockSpec((_RB, D), lambda i: (i, 0)),
            pl.BlockSpec((_RB, D), lambda i: (i, 0)),
            pl.BlockSpec((1, D), lambda i: (0, 0)),
        ],
        out_specs=pl.BlockSpec((_RB, D), lambda i: (i, 0)),
        out_shape=jax.ShapeDtypeStruct((N, D), jnp.float32),
    )(agg, hp, dinv, b2)


@jax.jit
def kernel(x, edge_index, W1, b1, W2, b2):
    src = edge_index[0].reshape(NW, NBLK, BLK)
    dst = edge_index[1].reshape(NW, NBLK, BLK)
    ones16 = jnp.ones((BLK, 16), jnp.float32)
    zer16 = jnp.zeros((ZCH, 16), jnp.float32)
    zer128 = jnp.zeros((ZCH, D), jnp.float32)
    b1r = b1.reshape(1, D)
    b2r = b2.reshape(1, D)

    degp = _sc_degree(dst, ones16, zer16)
    hp1, dinv = _tc_prescale(degp, x, W1)
    agg1 = _sc_scatter(hp1, src, dst, zer128)
    hp2 = _tc_mid(agg1, hp1, dinv, b1r, W2)
    agg2 = _sc_scatter(hp2, src, dst, zer128)
    return _tc_final(agg2, hp2, dinv, b2r)


# speedup vs baseline: 14.3304x; 14.3304x over previous
import functools

import jax
import jax.numpy as jnp
from jax import lax
from jax.experimental import pallas as pl
from jax.experimental.pallas import tpu as pltpu
from jax.experimental.pallas import tpu_sc as plsc

N = 10000
E = 320000
D = 128
DH = D // 2

NC = 2
NS = 16
NW = NC * NS
BLK = 80
DBLK = NW * BLK
DNBLK = E // DBLK
SBLK = NS * BLK
SNBLK = E // SBLK
NPAD = 10240
RPT = NPAD // NS
ZCH = 128

_MESH = plsc.VectorSubcoreMesh(
    core_axis_name="c", subcore_axis_name="s", num_cores=NC, num_subcores=NS
)


@functools.partial(
    pl.kernel,
    out_type=jax.ShapeDtypeStruct((NC, NPAD, 16), jnp.float32),
    mesh=_MESH,
    compiler_params=pltpu.CompilerParams(use_tc_tiling_on_sc=False),
    scratch_types=[
        pltpu.VMEM((DNBLK, BLK), jnp.int32),
        pltpu.VMEM((BLK, 16), jnp.float32),
        pltpu.VMEM((ZCH, 16), jnp.float32),
        pltpu.VMEM_SHARED((NPAD, 16), jnp.float32),
    ],
)
def _sc_degree(dst_hbm, ones_hbm, zer_hbm, out_hbm, dst_v, ones_v, zer_v, acc):
    cid = lax.axis_index("c")
    sid = lax.axis_index("s")
    wid = cid * NS + sid
    pltpu.sync_copy(dst_hbm.at[wid], dst_v)
    pltpu.sync_copy(ones_hbm, ones_v)
    pltpu.sync_copy(zer_hbm, zer_v)
    row0 = sid * RPT
    for k in range(RPT // ZCH):
        pltpu.sync_copy(zer_v, acc.at[pl.ds(row0 + k * ZCH, ZCH)])
    plsc.subcore_barrier()

    def body(j, carry):
        pltpu.sync_copy(ones_v, acc.at[dst_v.at[j]], add=True)
        return carry

    lax.fori_loop(0, DNBLK, body, 0)
    plsc.subcore_barrier()
    pltpu.sync_copy(acc.at[pl.ds(row0, RPT)], out_hbm.at[cid, pl.ds(row0, RPT)])


@functools.partial(
    pl.kernel,
    out_type=jax.ShapeDtypeStruct((NC, NPAD, DH), jnp.float32),
    mesh=_MESH,
    compiler_params=pltpu.CompilerParams(use_tc_tiling_on_sc=False),
    scratch_types=[
        pltpu.VMEM((SNBLK, BLK), jnp.int32),
        pltpu.VMEM((SNBLK, BLK), jnp.int32),
        pltpu.VMEM((BLK, DH), jnp.float32),
        pltpu.VMEM((ZCH, DH), jnp.float32),
        pltpu.VMEM_SHARED((NPAD, DH), jnp.float32),
        pltpu.SemaphoreType.DMA,
    ],
)
def _sc_scatter(h_hbm, src_hbm, dst_hbm, zer_hbm, out_hbm,
                src_v, dst_v, buf, zer_v, acc, sem):
    cid = lax.axis_index("c")
    sid = lax.axis_index("s")
    pltpu.sync_copy(src_hbm.at[sid], src_v)
    pltpu.sync_copy(dst_hbm.at[sid], dst_v)
    pltpu.sync_copy(zer_hbm, zer_v)
    row0 = sid * RPT
    for k in range(RPT // ZCH):
        pltpu.sync_copy(zer_v, acc.at[pl.ds(row0 + k * ZCH, ZCH)])
    plsc.subcore_barrier()
    htab = h_hbm.at[cid]

    def body(j, carry):
        pltpu.async_copy(htab.at[src_v.at[j]], buf, sem).wait()
        pltpu.sync_copy(buf, acc.at[dst_v.at[j]], add=True)
        return carry

    lax.fori_loop(0, SNBLK, body, 0)
    plsc.subcore_barrier()
    pltpu.sync_copy(acc.at[pl.ds(row0, RPT)], out_hbm.at[cid, pl.ds(row0, RPT)])


_RB = 1000
_GRID = N // _RB


def _split(h):
    return jnp.stack([h[:, :DH], h[:, DH:]], axis=0)


def _tc_prescale_body(degp_ref, x_ref, w_ref, hp_ref, dinv_ref):
    d = degp_ref[...]
    deg = d[0, :, 0] + d[1, :, 0] + 1.0
    dinv = lax.rsqrt(deg)
    h = jnp.dot(x_ref[...], w_ref[...], preferred_element_type=jnp.float32)
    hp_ref[...] = _split(h * dinv[:, None])
    dinv_ref[...] = jnp.broadcast_to(dinv[:, None], (_RB, D))


def _tc_prescale(degp, x, w1):
    return pl.pallas_call(
        _tc_prescale_body,
        grid=(_GRID,),
        in_specs=[
            pl.BlockSpec((NC, _RB, 16), lambda i: (0, i, 0)),
            pl.BlockSpec((_RB, D), lambda i: (i, 0)),
            pl.BlockSpec((D, D), lambda i: (0, 0)),
        ],
        out_specs=[
            pl.BlockSpec((NC, _RB, DH), lambda i: (0, i, 0)),
            pl.BlockSpec((_RB, D), lambda i: (i, 0)),
        ],
        out_shape=[
            jax.ShapeDtypeStruct((NC, N, DH), jnp.float32),
            jax.ShapeDtypeStruct((N, D), jnp.float32),
        ],
    )(degp, x, w1)


def _tc_mid_body(agg_ref, hp_ref, dinv_ref, b_ref, w_ref, hp2_ref):
    s = agg_ref[...] + hp_ref[...]
    full = jnp.concatenate([s[0], s[1]], axis=1)
    dinv = dinv_ref[...]
    pre = dinv * full + b_ref[...]
    y = jnp.where(pre > 0, pre, jnp.exp(pre) - 1.0)
    h2 = jnp.dot(y, w_ref[...], preferred_element_type=jnp.float32)
    hp2_ref[...] = _split(h2 * dinv)


def _tc_mid(agg, hp, dinv, b1, w2):
    return pl.pallas_call(
        _tc_mid_body,
        grid=(_GRID,),
        in_specs=[
            pl.BlockSpec((NC, _RB, DH), lambda i: (0, i, 0)),
            pl.BlockSpec((NC, _RB, DH), lambda i: (0, i, 0)),
            pl.BlockSpec((_RB, D), lambda i: (i, 0)),
            pl.BlockSpec((1, D), lambda i: (0, 0)),
            pl.BlockSpec((D, D), lambda i: (0, 0)),
        ],
        out_specs=pl.BlockSpec((NC, _RB, DH), lambda i: (0, i, 0)),
        out_shape=jax.ShapeDtypeStruct((NC, N, DH), jnp.float32),
    )(agg, hp, dinv, b1, w2)


def _tc_final_body(agg_ref, hp_ref, dinv_ref, b_ref, y_ref):
    s = agg_ref[...] + hp_ref[...]
    full = jnp.concatenate([s[0], s[1]], axis=1)
    pre = dinv_ref[...] * full + b_ref[...]
    y_ref[...] = jnp.where(pre > 0, pre, jnp.exp(pre) - 1.0)


def _tc_final(agg, hp, dinv, b2):
    return pl.pallas_call(
        _tc_final_body,
        grid=(_GRID,),
        in_specs=[
            pl.BlockSpec((NC, _RB, DH), lambda i: (0, i, 0)),
            pl.BlockSpec((NC, _RB, DH), lambda i: (0, i, 0)),
            pl.BlockSpec((_RB, D), lambda i: (i, 0)),
            pl.BlockSpec((1, D), lambda i: (0, 0)),
        ],
        out_specs=pl.BlockSpec((_RB, D), lambda i: (i, 0)),
        out_shape=jax.ShapeDtypeStruct((N, D), jnp.float32),
    )(agg, hp, dinv, b2)


@jax.jit
def kernel(x, edge_index, W1, b1, W2, b2):
    dst32 = edge_index[1].reshape(NW, DNBLK, BLK)
    src16 = edge_index[0].reshape(NS, SNBLK, BLK)
    dst16 = edge_index[1].reshape(NS, SNBLK, BLK)
    ones16 = jnp.ones((BLK, 16), jnp.float32)
    zer16 = jnp.zeros((ZCH, 16), jnp.float32)
    zer64 = jnp.zeros((ZCH, DH), jnp.float32)
    b1r = b1.reshape(1, D)
    b2r = b2.reshape(1, D)

    degp = _sc_degree(dst32, ones16, zer16)
    hp1, dinv = _tc_prescale(degp, x, W1)
    agg1 = _sc_scatter(hp1, src16, dst16, zer64)
    hp2 = _tc_mid(agg1, hp1, dinv, b1r, W2)
    agg2 = _sc_scatter(hp2, src16, dst16, zer64)
    return _tc_final(agg2, hp2, dinv, b2r)


# speedup vs baseline: 24.1160x; 1.6829x over previous
import functools

import jax
import jax.numpy as jnp
from jax import lax
from jax.experimental import pallas as pl
from jax.experimental.pallas import tpu as pltpu
from jax.experimental.pallas import tpu_sc as plsc

N = 10000
E = 320000
D = 128
DH = D // 2

NC = 2
NS = 16
NW = NC * NS
BLK = 100
DNBLK = E // (NW * BLK)
SNBLK = E // (NS * BLK)
NPAD = 10240
RPT = NPAD // NS
ZCH = 80
M = 5
LA = 3

_MESH = plsc.VectorSubcoreMesh(
    core_axis_name="c", subcore_axis_name="s", num_cores=NC, num_subcores=NS
)


@functools.partial(
    pl.kernel,
    out_type=jax.ShapeDtypeStruct((NC, NPAD, 16), jnp.float32),
    mesh=_MESH,
    compiler_params=pltpu.CompilerParams(use_tc_tiling_on_sc=False),
    scratch_types=[
        pltpu.VMEM((DNBLK, BLK), jnp.int32),
        pltpu.VMEM((BLK, 16), jnp.float32),
        pltpu.VMEM((ZCH, 16), jnp.float32),
        pltpu.VMEM_SHARED((NPAD, 16), jnp.float32),
    ],
)
def _sc_degree(dst_hbm, ones_hbm, zer_hbm, out_hbm, dst_v, ones_v, zer_v, acc):
    cid = lax.axis_index("c")
    sid = lax.axis_index("s")
    wid = cid * NS + sid
    pltpu.sync_copy(dst_hbm.at[wid], dst_v)
    pltpu.sync_copy(ones_hbm, ones_v)
    pltpu.sync_copy(zer_hbm, zer_v)
    row0 = sid * RPT
    for k in range(RPT // ZCH):
        pltpu.sync_copy(zer_v, acc.at[pl.ds(row0 + k * ZCH, ZCH)])
    plsc.subcore_barrier()

    def body(j, carry):
        pltpu.sync_copy(ones_v, acc.at[dst_v.at[j]], add=True)
        return carry

    lax.fori_loop(0, DNBLK, body, 0)
    plsc.subcore_barrier()
    pltpu.sync_copy(acc.at[pl.ds(row0, RPT)], out_hbm.at[cid, pl.ds(row0, RPT)])


@functools.partial(
    pl.kernel,
    out_type=jax.ShapeDtypeStruct((NC, NPAD, DH), jnp.float32),
    mesh=_MESH,
    compiler_params=pltpu.CompilerParams(use_tc_tiling_on_sc=False),
    scratch_types=[
        pltpu.VMEM((SNBLK, BLK), jnp.int32),
        pltpu.VMEM((SNBLK, BLK), jnp.int32),
        pltpu.VMEM((M, BLK, DH), jnp.float32),
        pltpu.VMEM((ZCH, DH), jnp.float32),
        pltpu.VMEM_SHARED((NPAD, DH), jnp.float32),
    ] + [pltpu.SemaphoreType.DMA] * (2 * M),
)
def _sc_scatter(h_hbm, src_hbm, dst_hbm, zer_hbm, out_hbm,
                src_v, dst_v, bufs, zer_v, acc, *sems):
    gsems = sems[:M]
    ssems = sems[M:]
    cid = lax.axis_index("c")
    sid = lax.axis_index("s")
    pltpu.sync_copy(src_hbm.at[sid], src_v)
    pltpu.sync_copy(dst_hbm.at[sid], dst_v)
    pltpu.sync_copy(zer_hbm, zer_v)
    row0 = sid * RPT
    for k in range(RPT // ZCH):
        pltpu.sync_copy(zer_v, acc.at[pl.ds(row0 + k * ZCH, ZCH)])
    plsc.subcore_barrier()
    htab = h_hbm.at[cid]

    def group(g, carry):
        gds = [pltpu.async_copy(htab.at[src_v.at[g * M + b]], bufs.at[b],
                                gsems[b]) for b in range(M)]
        sds = []
        for b in range(M):
            gds[b].wait()
            sds.append(pltpu.async_copy(bufs.at[b], acc.at[dst_v.at[g * M + b]],
                                        ssems[b], add=True))
        for b in range(M):
            sds[b].wait()
        return carry

    lax.fori_loop(0, SNBLK // M, group, 0)
    plsc.subcore_barrier()
    pltpu.sync_copy(acc.at[pl.ds(row0, RPT)], out_hbm.at[cid, pl.ds(row0, RPT)])


_RB = 1000
_GRID = N // _RB


def _split(h):
    return jnp.stack([h[:, :DH], h[:, DH:]], axis=0)


def _tc_prescale_body(degp_ref, x_ref, w_ref, hp_ref, dinv_ref):
    d = degp_ref[...]
    deg = d[0, :, 0] + d[1, :, 0] + 1.0
    dinv = lax.rsqrt(deg)
    h = jnp.dot(x_ref[...], w_ref[...], preferred_element_type=jnp.float32)
    hp_ref[...] = _split(h * dinv[:, None])
    dinv_ref[...] = jnp.broadcast_to(dinv[:, None], (_RB, D))


def _tc_prescale(degp, x, w1):
    return pl.pallas_call(
        _tc_prescale_body,
        grid=(_GRID,),
        in_specs=[
            pl.BlockSpec((NC, _RB, 16), lambda i: (0, i, 0)),
            pl.BlockSpec((_RB, D), lambda i: (i, 0)),
            pl.BlockSpec((D, D), lambda i: (0, 0)),
        ],
        out_specs=[
            pl.BlockSpec((NC, _RB, DH), lambda i: (0, i, 0)),
            pl.BlockSpec((_RB, D), lambda i: (i, 0)),
        ],
        out_shape=[
            jax.ShapeDtypeStruct((NC, N, DH), jnp.float32),
            jax.ShapeDtypeStruct((N, D), jnp.float32),
        ],
    )(degp, x, w1)


def _tc_mid_body(agg_ref, hp_ref, dinv_ref, b_ref, w_ref, hp2_ref):
    s = agg_ref[...] + hp_ref[...]
    full = jnp.concatenate([s[0], s[1]], axis=1)
    dinv = dinv_ref[...]
    pre = dinv * full + b_ref[...]
    y = jnp.where(pre > 0, pre, jnp.exp(pre) - 1.0)
    h2 = jnp.dot(y, w_ref[...], preferred_element_type=jnp.float32)
    hp2_ref[...] = _split(h2 * dinv)


def _tc_mid(agg, hp, dinv, b1, w2):
    return pl.pallas_call(
        _tc_mid_body,
        grid=(_GRID,),
        in_specs=[
            pl.BlockSpec((NC, _RB, DH), lambda i: (0, i, 0)),
            pl.BlockSpec((NC, _RB, DH), lambda i: (0, i, 0)),
            pl.BlockSpec((_RB, D), lambda i: (i, 0)),
            pl.BlockSpec((1, D), lambda i: (0, 0)),
            pl.BlockSpec((D, D), lambda i: (0, 0)),
        ],
        out_specs=pl.BlockSpec((NC, _RB, DH), lambda i: (0, i, 0)),
        out_shape=jax.ShapeDtypeStruct((NC, N, DH), jnp.float32),
    )(agg, hp, dinv, b1, w2)


def _tc_final_body(agg_ref, hp_ref, dinv_ref, b_ref, y_ref):
    s = agg_ref[...] + hp_ref[...]
    full = jnp.concatenate([s[0], s[1]], axis=1)
    pre = dinv_ref[...] * full + b_ref[...]
    y_ref[...] = jnp.where(pre > 0, pre, jnp.exp(pre) - 1.0)


def _tc_final(agg, hp, dinv, b2):
    return pl.pallas_call(
        _tc_final_body,
        grid=(_GRID,),
        in_specs=[
            pl.BlockSpec((NC, _RB, DH), lambda i: (0, i, 0)),
            pl.BlockSpec((NC, _RB, DH), lambda i: (0, i, 0)),
            pl.BlockSpec((_RB, D), lambda i: (i, 0)),
            pl.BlockSpec((1, D), lambda i: (0, 0)),
        ],
        out_specs=pl.BlockSpec((_RB, D), lambda i: (i, 0)),
        out_shape=jax.ShapeDtypeStruct((N, D), jnp.float32),
    )(agg, hp, dinv, b2)


@jax.jit
def kernel(x, edge_index, W1, b1, W2, b2):
    dst32 = edge_index[1].reshape(NW, DNBLK, BLK)
    src16 = edge_index[0].reshape(NS, SNBLK, BLK)
    dst16 = edge_index[1].reshape(NS, SNBLK, BLK)
    ones16 = jnp.ones((BLK, 16), jnp.float32)
    zer16 = jnp.zeros((ZCH, 16), jnp.float32)
    zer64 = jnp.zeros((ZCH, DH), jnp.float32)
    b1r = b1.reshape(1, D)
    b2r = b2.reshape(1, D)

    degp = _sc_degree(dst32, ones16, zer16)
    hp1, dinv = _tc_prescale(degp, x, W1)
    agg1 = _sc_scatter(hp1, src16, dst16, zer64)
    hp2 = _tc_mid(agg1, hp1, dinv, b1r, W2)
    agg2 = _sc_scatter(hp2, src16, dst16, zer64)
    return _tc_final(agg2, hp2, dinv, b2r)


# speedup vs baseline: 25.5134x; 1.0579x over previous
import functools

import jax
import jax.numpy as jnp
from jax import lax
from jax.experimental import pallas as pl
from jax.experimental.pallas import tpu as pltpu
from jax.experimental.pallas import tpu_sc as plsc

N = 10000
E = 320000
D = 128
DH = D // 2

NC = 2
NS = 16
NW = NC * NS
BLK = 125
DNBLK = E // (NW * BLK)
SNBLK = E // (NS * BLK)
NPAD = 10240
RPT = NPAD // NS
ZCH = 80
M = 8
SUP = 4
NSUP = SNBLK // (SUP * M)

_MESH = plsc.VectorSubcoreMesh(
    core_axis_name="c", subcore_axis_name="s", num_cores=NC, num_subcores=NS
)


@functools.partial(
    pl.kernel,
    out_type=jax.ShapeDtypeStruct((NC, NPAD, 16), jnp.float32),
    mesh=_MESH,
    compiler_params=pltpu.CompilerParams(use_tc_tiling_on_sc=False),
    scratch_types=[
        pltpu.VMEM((DNBLK, BLK), jnp.int32),
        pltpu.VMEM((BLK, 16), jnp.float32),
        pltpu.VMEM((ZCH, 16), jnp.float32),
        pltpu.VMEM_SHARED((NPAD, 16), jnp.float32),
    ] + [pltpu.SemaphoreType.DMA] * M,
)
def _sc_degree(dst_hbm, ones_hbm, zer_hbm, out_hbm, dst_v, ones_v, zer_v, acc,
               *sems):
    cid = lax.axis_index("c")
    sid = lax.axis_index("s")
    wid = cid * NS + sid
    pltpu.sync_copy(dst_hbm.at[wid], dst_v)
    pltpu.sync_copy(ones_hbm, ones_v)
    pltpu.sync_copy(zer_hbm, zer_v)
    row0 = sid * RPT
    zds = [pltpu.async_copy(zer_v, acc.at[pl.ds(row0 + k * ZCH, ZCH)], sems[k])
           for k in range(RPT // ZCH)]
    for d in zds:
        d.wait()
    plsc.subcore_barrier()

    def body(g, carry):
        sds = [pltpu.async_copy(ones_v, acc.at[dst_v.at[g * M + b]], sems[b],
                                add=True) for b in range(M)]
        for d in sds:
            d.wait()
        return carry

    lax.fori_loop(0, DNBLK // M, body, 0)
    plsc.subcore_barrier()
    pltpu.sync_copy(acc.at[pl.ds(row0, RPT)], out_hbm.at[cid, pl.ds(row0, RPT)])


@functools.partial(
    pl.kernel,
    out_type=jax.ShapeDtypeStruct((NC, NPAD, DH), jnp.float32),
    mesh=_MESH,
    compiler_params=pltpu.CompilerParams(use_tc_tiling_on_sc=False),
    scratch_types=[
        pltpu.VMEM((SUP, M, 2, BLK), jnp.int32),
        pltpu.VMEM((M, BLK, DH), jnp.float32),
        pltpu.VMEM((ZCH, DH), jnp.float32),
        pltpu.VMEM_SHARED((NPAD, DH), jnp.float32),
    ] + [pltpu.SemaphoreType.DMA] * (2 * M),
)
def _sc_scatter(h_hbm, idx_hbm, zer_hbm, out_hbm,
                ichunk, bufs, zer_v, acc, *sems):
    gsems = sems[:M]
    ssems = sems[M:]
    cid = lax.axis_index("c")
    sid = lax.axis_index("s")
    pltpu.sync_copy(zer_hbm, zer_v)
    row0 = sid * RPT
    zds = [pltpu.async_copy(zer_v, acc.at[pl.ds(row0 + k * ZCH, ZCH)],
                            gsems[k]) for k in range(RPT // ZCH)]
    for d in zds:
        d.wait()
    plsc.subcore_barrier()
    htab = h_hbm.at[cid]

    def super_body(sb, carry):
        pltpu.sync_copy(idx_hbm.at[sid, sb], ichunk)
        for g in range(SUP):
            gds = [pltpu.async_copy(htab.at[ichunk.at[g, b, 0]], bufs.at[b],
                                    gsems[b]) for b in range(M)]
            sds = []
            for b in range(M):
                gds[b].wait()
                sds.append(pltpu.async_copy(bufs.at[b],
                                            acc.at[ichunk.at[g, b, 1]],
                                            ssems[b], add=True))
            for b in range(M):
                sds[b].wait()
        return carry

    lax.fori_loop(0, NSUP, super_body, 0)
    plsc.subcore_barrier()
    pltpu.sync_copy(acc.at[pl.ds(row0, RPT)], out_hbm.at[cid, pl.ds(row0, RPT)])


_RB = 1000
_GRID = N // _RB


def _split(h):
    return jnp.stack([h[:, :DH], h[:, DH:]], axis=0)


def _tc_prescale_body(degp_ref, x_ref, w_ref, hp_ref, dinv_ref):
    d = degp_ref[...]
    deg = d[0, :, 0] + d[1, :, 0] + 1.0
    dinv = lax.rsqrt(deg)
    h = jnp.dot(x_ref[...], w_ref[...], preferred_element_type=jnp.float32)
    hp_ref[...] = _split(h * dinv[:, None])
    dinv_ref[...] = jnp.broadcast_to(dinv[:, None], (_RB, D))


def _tc_prescale(degp, x, w1):
    return pl.pallas_call(
        _tc_prescale_body,
        grid=(_GRID,),
        in_specs=[
            pl.BlockSpec((NC, _RB, 16), lambda i: (0, i, 0)),
            pl.BlockSpec((_RB, D), lambda i: (i, 0)),
            pl.BlockSpec((D, D), lambda i: (0, 0)),
        ],
        out_specs=[
            pl.BlockSpec((NC, _RB, DH), lambda i: (0, i, 0)),
            pl.BlockSpec((_RB, D), lambda i: (i, 0)),
        ],
        out_shape=[
            jax.ShapeDtypeStruct((NC, N, DH), jnp.float32),
            jax.ShapeDtypeStruct((N, D), jnp.float32),
        ],
    )(degp, x, w1)


def _tc_mid_body(agg_ref, hp_ref, dinv_ref, b_ref, w_ref, hp2_ref):
    s = agg_ref[...] + hp_ref[...]
    full = jnp.concatenate([s[0], s[1]], axis=1)
    dinv = dinv_ref[...]
    pre = dinv * full + b_ref[...]
    y = jnp.where(pre > 0, pre, jnp.exp(pre) - 1.0)
    h2 = jnp.dot(y, w_ref[...], preferred_element_type=jnp.float32)
    hp2_ref[...] = _split(h2 * dinv)


def _tc_mid(agg, hp, dinv, b1, w2):
    return pl.pallas_call(
        _tc_mid_body,
        grid=(_GRID,),
        in_specs=[
            pl.BlockSpec((NC, _RB, DH), lambda i: (0, i, 0)),
            pl.BlockSpec((NC, _RB, DH), lambda i: (0, i, 0)),
            pl.BlockSpec((_RB, D), lambda i: (i, 0)),
            pl.BlockSpec((1, D), lambda i: (0, 0)),
            pl.BlockSpec((D, D), lambda i: (0, 0)),
        ],
        out_specs=pl.BlockSpec((NC, _RB, DH), lambda i: (0, i, 0)),
        out_shape=jax.ShapeDtypeStruct((NC, N, DH), jnp.float32),
    )(agg, hp, dinv, b1, w2)


def _tc_final_body(agg_ref, hp_ref, dinv_ref, b_ref, y_ref):
    s = agg_ref[...] + hp_ref[...]
    full = jnp.concatenate([s[0], s[1]], axis=1)
    pre = dinv_ref[...] * full + b_ref[...]
    y_ref[...] = jnp.where(pre > 0, pre, jnp.exp(pre) - 1.0)


def _tc_final(agg, hp, dinv, b2):
    return pl.pallas_call(
        _tc_final_body,
        grid=(_GRID,),
        in_specs=[
            pl.BlockSpec((NC, _RB, DH), lambda i: (0, i, 0)),
            pl.BlockSpec((NC, _RB, DH), lambda i: (0, i, 0)),
            pl.BlockSpec((_RB, D), lambda i: (i, 0)),
            pl.BlockSpec((1, D), lambda i: (0, 0)),
        ],
        out_specs=pl.BlockSpec((_RB, D), lambda i: (i, 0)),
        out_shape=jax.ShapeDtypeStruct((N, D), jnp.float32),
    )(agg, hp, dinv, b2)


@jax.jit
def kernel(x, edge_index, W1, b1, W2, b2):
    dst32 = edge_index[1].reshape(NW, DNBLK, BLK)
    src16 = edge_index[0].reshape(NS, SNBLK, BLK)
    dst16 = edge_index[1].reshape(NS, SNBLK, BLK)
    idx16 = jnp.stack([src16, dst16], axis=2).reshape(
        NS, NSUP, SUP, M, 2, BLK)
    ones16 = jnp.ones((BLK, 16), jnp.float32)
    zer16 = jnp.zeros((ZCH, 16), jnp.float32)
    zer64 = jnp.zeros((ZCH, DH), jnp.float32)
    b1r = b1.reshape(1, D)
    b2r = b2.reshape(1, D)

    degp = _sc_degree(dst32, ones16, zer16)
    hp1, dinv = _tc_prescale(degp, x, W1)
    agg1 = _sc_scatter(hp1, idx16, zer64)
    hp2 = _tc_mid(agg1, hp1, dinv, b1r, W2)
    agg2 = _sc_scatter(hp2, idx16, zer64)
    return _tc_final(agg2, hp2, dinv, b2r)
